# baseline (speedup 1.0000x reference)
import numpy as np

import jax
import jax.numpy as jnp
from jax.experimental import pallas as pl
from jax.experimental.pallas import tpu as pltpu

TB = 512


def _build_s1():
    s = np.zeros((140, 2, 12, 25), np.float32)
    for ty in range(5):
        for tx in range(5):
            for j in range(2):
                for B in range(12):
                    s[ty * 28 + (2 * B + j + tx), j, B, ty * 5 + tx] = 1.0
    return s


def _build_s2():
    s = np.zeros((2, 4, 12, 5), np.float32)
    for kx in range(5):
        for j in range(2):
            for B2 in range(4):
                s[j, B2, 2 * B2 + j + kx, kx] = 1.0
    return s


_S1 = _build_s1()
_S2 = _build_s2()


def _net_kernel(x_ref, w1_ref, b1_ref, w2_ref, b2_ref,
                wf1_ref, bf1_ref, wf2_ref, bf2_ref,
                out_ref, xf_ref, p1_ref):
    f32 = jnp.float32
    bf16 = jnp.bfloat16
    w1 = w1_ref[...]

    xf_ref[...] = x_ref[...].astype(bf16).reshape(TB, 784)

    for Ap in range(6):
        o = jnp.dot(xf_ref[:, 112 * Ap: 112 * Ap + 224], w1,
                    preferred_element_type=f32)
        m0 = jnp.maximum(jnp.maximum(o[:, 0:128], o[:, 128:256]),
                         jnp.maximum(o[:, 256:384], o[:, 384:512]))
        m1 = jnp.maximum(jnp.maximum(o[:, 512:640], o[:, 640:768]),
                         jnp.maximum(o[:, 768:896], o[:, 896:1024]))
        p1_ref[:, 256 * Ap: 256 * Ap + 128] = jnp.maximum(
            m0 + b1_ref[...], 0.0).astype(bf16)
        p1_ref[:, 256 * Ap + 128: 256 * Ap + 256] = jnp.maximum(
            m1 + b1_ref[...], 0.0).astype(bf16)

    w2 = w2_ref[...]
    flat_parts = []
    for A2 in range(4):
        o = jnp.dot(p1_ref[:, 256 * A2: 256 * A2 + 768], w2,
                    preferred_element_type=f32)
        m = jnp.maximum(jnp.maximum(o[:, 0:128], o[:, 128:256]),
                        jnp.maximum(o[:, 256:384], o[:, 384:512]))
        flat_parts.append(jnp.maximum(m + b2_ref[...], 0.0).astype(bf16))
    flat = jnp.concatenate(flat_parts, axis=1)

    h = jnp.maximum(jnp.dot(flat, wf1_ref[...], preferred_element_type=f32)
                    + bf1_ref[...], 0.0).astype(bf16)
    logits = jnp.dot(h, wf2_ref[...], preferred_element_type=f32) + bf2_ref[...]
    s = logits - jnp.max(logits, axis=1, keepdims=True)
    out_ref[...] = s - jnp.log(jnp.sum(jnp.exp(s), axis=1, keepdims=True))


def kernel(x, w1t, b1, w2m, b2, wf1, bf1, wf2, bf2):
    n = x.shape[0]
    n_pad = ((n + TB - 1) // TB) * TB
    x2 = x.astype(jnp.float32).reshape(n, 28, 28)
    if n_pad != n:
        x2 = jnp.pad(x2, ((0, n_pad - n), (0, 0), (0, 0)))

    f32 = jnp.float32
    bf16 = jnp.bfloat16
    t1 = jnp.einsum('kjbt,tc->kjbc', _S1, w1t.reshape(25, 10))
    w1m = jnp.pad(t1.reshape(140, 2, 120),
                  ((0, 0), (0, 0), (0, 8))).reshape(140, 256)
    w1m = jnp.concatenate(
        [jnp.pad(w1m, ((28 * g, 84 - 28 * g), (0, 0))) for g in range(4)],
        axis=1).astype(bf16)
    w2r = w2m.reshape(20, 5, 5, 10)
    t2 = jnp.einsum('jqbx,oyxi->ybijqo', _S2, w2r)
    w2big = jnp.pad(t2.reshape(5, 120, 2, 80),
                    ((0, 0), (0, 8), (0, 0), (0, 48))).reshape(640, 256)
    w2big = jnp.concatenate(
        [jnp.pad(w2big, ((0, 128), (0, 0))),
         jnp.pad(w2big, ((128, 0), (0, 0)))], axis=1).astype(bf16)
    wf1m = jnp.pad(wf1.T.reshape(4, 80, 50),
                   ((0, 0), (0, 48), (0, 0))).reshape(512, 50).astype(bf16)
    b1l = jnp.pad(jnp.tile(b1.reshape(10), 12), (0, 8))[None, :]
    b2l = jnp.pad(jnp.tile(b2.reshape(20), 4), (0, 48))[None, :]
    bf1l = bf1.reshape(1, 50)
    bf2l = bf2.reshape(1, 10)
    wf2m = wf2.T.astype(bf16)

    out = pl.pallas_call(
        _net_kernel,
        out_shape=jax.ShapeDtypeStruct((n_pad, 10), jnp.float32),
        grid_spec=pltpu.PrefetchScalarGridSpec(
            num_scalar_prefetch=0,
            grid=(n_pad // TB,),
            in_specs=[
                pl.BlockSpec((TB, 28, 28), lambda t: (t, 0, 0)),
                pl.BlockSpec((224, 1024), lambda t: (0, 0)),
                pl.BlockSpec((1, 128), lambda t: (0, 0)),
                pl.BlockSpec((768, 512), lambda t: (0, 0)),
                pl.BlockSpec((1, 128), lambda t: (0, 0)),
                pl.BlockSpec((512, 50), lambda t: (0, 0)),
                pl.BlockSpec((1, 50), lambda t: (0, 0)),
                pl.BlockSpec((50, 10), lambda t: (0, 0)),
                pl.BlockSpec((1, 10), lambda t: (0, 0)),
            ],
            out_specs=pl.BlockSpec((TB, 10), lambda t: (t, 0)),
            scratch_shapes=[
                pltpu.VMEM((TB, 784), jnp.bfloat16),
                pltpu.VMEM((TB, 12 * 128), jnp.bfloat16),
            ],
        ),
        compiler_params=pltpu.CompilerParams(
            dimension_semantics=("parallel",),
            vmem_limit_bytes=64 * 1024 * 1024,
        ),
    )(x2, w1m, b1l, w2big, b2l, wf1m, bf1l, wf2m, bf2l)
    return out[:n]


# speedup vs baseline: 1.0155x; 1.0155x over previous
import numpy as np

import jax
import jax.numpy as jnp
from jax.experimental import pallas as pl
from jax.experimental.pallas import tpu as pltpu

TB = 1024


def _build_s1():
    s = np.zeros((140, 2, 12, 25), np.float32)
    for ty in range(5):
        for tx in range(5):
            for j in range(2):
                for B in range(12):
                    s[ty * 28 + (2 * B + j + tx), j, B, ty * 5 + tx] = 1.0
    return s


def _build_s2():
    s = np.zeros((2, 4, 12, 5), np.float32)
    for kx in range(5):
        for j in range(2):
            for B2 in range(4):
                s[j, B2, 2 * B2 + j + kx, kx] = 1.0
    return s


_S1 = _build_s1()
_S2 = _build_s2()


def _net_kernel(x_ref, w1_ref, b1_ref, w2_ref, b2_ref,
                wf1_ref, bf1_ref, wf2_ref, bf2_ref,
                out_ref, xf_ref, p1_ref):
    f32 = jnp.float32
    bf16 = jnp.bfloat16
    w1 = w1_ref[...]

    xf_ref[...] = x_ref[...].astype(bf16).reshape(TB, 784)

    for Ap in range(6):
        o = jnp.dot(xf_ref[:, 112 * Ap: 112 * Ap + 224], w1,
                    preferred_element_type=f32)
        m0 = jnp.maximum(jnp.maximum(o[:, 0:128], o[:, 128:256]),
                         jnp.maximum(o[:, 256:384], o[:, 384:512]))
        m1 = jnp.maximum(jnp.maximum(o[:, 512:640], o[:, 640:768]),
                         jnp.maximum(o[:, 768:896], o[:, 896:1024]))
        p1_ref[:, 256 * Ap: 256 * Ap + 128] = jnp.maximum(
            m0 + b1_ref[...], 0.0).astype(bf16)
        p1_ref[:, 256 * Ap + 128: 256 * Ap + 256] = jnp.maximum(
            m1 + b1_ref[...], 0.0).astype(bf16)

    w2 = w2_ref[...]
    flat_parts = []
    for A2 in range(4):
        o = jnp.dot(p1_ref[:, 256 * A2: 256 * A2 + 768], w2,
                    preferred_element_type=f32)
        m = jnp.maximum(jnp.maximum(o[:, 0:128], o[:, 128:256]),
                        jnp.maximum(o[:, 256:384], o[:, 384:512]))
        flat_parts.append(jnp.maximum(m + b2_ref[...], 0.0).astype(bf16))
    flat = jnp.concatenate(flat_parts, axis=1)

    h = jnp.maximum(jnp.dot(flat, wf1_ref[...], preferred_element_type=f32)
                    + bf1_ref[...], 0.0).astype(bf16)
    logits = jnp.dot(h, wf2_ref[...], preferred_element_type=f32) + bf2_ref[...]
    s = logits - jnp.max(logits, axis=1, keepdims=True)
    out_ref[...] = s - jnp.log(jnp.sum(jnp.exp(s), axis=1, keepdims=True))


def kernel(x, w1t, b1, w2m, b2, wf1, bf1, wf2, bf2):
    n = x.shape[0]
    n_pad = ((n + TB - 1) // TB) * TB
    x2 = x.astype(jnp.float32).reshape(n, 28, 28)
    if n_pad != n:
        x2 = jnp.pad(x2, ((0, n_pad - n), (0, 0), (0, 0)))

    f32 = jnp.float32
    bf16 = jnp.bfloat16
    t1 = jnp.einsum('kjbt,tc->kjbc', _S1, w1t.reshape(25, 10))
    w1m = jnp.pad(t1.reshape(140, 2, 120),
                  ((0, 0), (0, 0), (0, 8))).reshape(140, 256)
    w1m = jnp.concatenate(
        [jnp.pad(w1m, ((28 * g, 84 - 28 * g), (0, 0))) for g in range(4)],
        axis=1).astype(bf16)
    w2r = w2m.reshape(20, 5, 5, 10)
    t2 = jnp.einsum('jqbx,oyxi->ybijqo', _S2, w2r)
    w2big = jnp.pad(t2.reshape(5, 120, 2, 80),
                    ((0, 0), (0, 8), (0, 0), (0, 48))).reshape(640, 256)
    w2big = jnp.concatenate(
        [jnp.pad(w2big, ((0, 128), (0, 0))),
         jnp.pad(w2big, ((128, 0), (0, 0)))], axis=1).astype(bf16)
    wf1m = jnp.pad(wf1.T.reshape(4, 80, 50),
                   ((0, 0), (0, 48), (0, 0))).reshape(512, 50).astype(bf16)
    b1l = jnp.pad(jnp.tile(b1.reshape(10), 12), (0, 8))[None, :]
    b2l = jnp.pad(jnp.tile(b2.reshape(20), 4), (0, 48))[None, :]
    bf1l = bf1.reshape(1, 50)
    bf2l = bf2.reshape(1, 10)
    wf2m = wf2.T.astype(bf16)

    out = pl.pallas_call(
        _net_kernel,
        out_shape=jax.ShapeDtypeStruct((n_pad, 10), jnp.float32),
        grid_spec=pltpu.PrefetchScalarGridSpec(
            num_scalar_prefetch=0,
            grid=(n_pad // TB,),
            in_specs=[
                pl.BlockSpec((TB, 28, 28), lambda t: (t, 0, 0)),
                pl.BlockSpec((224, 1024), lambda t: (0, 0)),
                pl.BlockSpec((1, 128), lambda t: (0, 0)),
                pl.BlockSpec((768, 512), lambda t: (0, 0)),
                pl.BlockSpec((1, 128), lambda t: (0, 0)),
                pl.BlockSpec((512, 50), lambda t: (0, 0)),
                pl.BlockSpec((1, 50), lambda t: (0, 0)),
                pl.BlockSpec((50, 10), lambda t: (0, 0)),
                pl.BlockSpec((1, 10), lambda t: (0, 0)),
            ],
            out_specs=pl.BlockSpec((TB, 10), lambda t: (t, 0)),
            scratch_shapes=[
                pltpu.VMEM((TB, 784), jnp.bfloat16),
                pltpu.VMEM((TB, 12 * 128), jnp.bfloat16),
            ],
        ),
        compiler_params=pltpu.CompilerParams(
            dimension_semantics=("parallel",),
            vmem_limit_bytes=64 * 1024 * 1024,
        ),
    )(x2, w1m, b1l, w2big, b2l, wf1m, bf1l, wf2m, bf2l)
    return out[:n]
